# baseline (speedup 1.0000x reference)
import functools

import jax
import jax.numpy as jnp
from jax.experimental import pallas as pl
from jax.experimental.pallas import tpu as pltpu

_B, _S, _H, _RW, _C = 256, 3000, 64, 256, 5
_S2 = 2 * _S
_HT = 4 * _H
_CT = 4 * _C


def _fused_body(eeg_ref, eog_ref, w0_ref, w1_ref, w2_ref, w3_ref, rw_ref,
                b1_ref, w2bd_ref, b2_ref, rwb1_ref, rwW2_ref, rwb2_ref,
                rwWo_ref, rwbo_ref,
                eo_ref, w_ref, lg_ref, loss_ref):
    dot = functools.partial(jnp.dot, preferred_element_type=jnp.float32)
    eeg = eeg_ref[...]
    eog = eog_ref[...]
    etop = jnp.concatenate(
        [w0_ref[0:_S, :], w1_ref[0:_S, :], w2_ref[0:_S, :], w3_ref[0:_S, :]],
        axis=1)
    ebot = jnp.concatenate(
        [w0_ref[_S:_S2, :], w1_ref[_S:_S2, :], w2_ref[_S:_S2, :],
         w3_ref[_S:_S2, :]], axis=1)
    a = dot(eeg, etop)
    b = dot(eeg, ebot)
    c = dot(eog, etop)
    d = dot(eog, ebot)
    r = dot(eeg, rw_ref[0:_S, :]) + dot(eog, rw_ref[_S:_S2, :])

    b1 = b1_ref[...]
    h_ee = jnp.maximum(a + b + b1, 0.0)
    h_oo = jnp.maximum(c + d + b1, 0.0)
    h_eo = jnp.maximum(a + d + b1, 0.0)
    w2 = w2bd_ref[...]
    b2 = b2_ref[...]
    out_ee = dot(h_ee, w2) + b2
    out_oo = dot(h_oo, w2) + b2
    out_eo = dot(h_eo, w2) + b2
    eo_ref[0, :, :] = out_ee[:, 0:5]
    eo_ref[1, :, :] = out_oo[:, 5:10]
    eo_ref[2, :, :] = out_eo[:, 10:15]
    eo_ref[3, :, :] = out_eo[:, 15:20]

    hr = jnp.maximum(r + rwb1_ref[...], 0.0)
    h2 = jnp.maximum(dot(hr, rwW2_ref[...]) + rwb2_ref[...], 0.0)
    rlog = dot(h2, rwWo_ref[...]) + rwbo_ref[...]
    m = jnp.max(rlog, axis=-1, keepdims=True)
    ex = jnp.exp(rlog - m)
    wgt = ex / jnp.sum(ex, axis=-1, keepdims=True)
    w_ref[...] = wgt
    lg_ref[...] = (out_ee[:, 0:5] * wgt[:, 0:1]
                   + out_oo[:, 5:10] * wgt[:, 1:2]
                   + out_eo[:, 10:15] * wgt[:, 2:3]
                   + out_eo[:, 15:20] * wgt[:, 3:4])

    def _mse(x, y):
        dlt = x - y
        return jnp.mean(dlt * dlt)

    u_eeg = -_mse(out_ee[:, 0:5], out_oo[:, 0:5])
    u_eog = -_mse(out_oo[:, 5:10], out_ee[:, 5:10])
    syn = -_mse(out_eo[:, 10:15], 0.5 * (out_ee[:, 10:15] + out_oo[:, 10:15]))
    red = _mse(out_ee[:, 15:20], out_oo[:, 15:20])
    loss_ref[...] = jnp.concatenate(
        [u_eeg.reshape(1, 1), u_eog.reshape(1, 1),
         syn.reshape(1, 1), red.reshape(1, 1)], axis=1)


def kernel(eeg, eog,
           e0_W1, e0_b1, e0_W2, e0_b2,
           e1_W1, e1_b1, e1_W2, e1_b2,
           e2_W1, e2_b1, e2_W2, e2_b2,
           e3_W1, e3_b1, e3_W2, e3_b2,
           rw_W1, rw_b1, rw_W2, rw_b2, rw_Wo, rw_bo):
    bf = jnp.bfloat16
    in_vmem = functools.partial(
        pltpu.with_memory_space_constraint,
        memory_space=pltpu.MemorySpace.VMEM)
    eeg16 = eeg.astype(bf)
    eog16 = eog.astype(bf)
    w0 = in_vmem(e0_W1.astype(bf))
    w1 = e1_W1.astype(bf)
    w2_ = e2_W1.astype(bf)
    w3 = e3_W1.astype(bf)
    rw16 = in_vmem(rw_W1.astype(bf))
    b1_all = jnp.concatenate([e0_b1, e1_b1, e2_b1, e3_b1]).reshape(1, _HT)
    w2bd = jnp.zeros((_HT, _CT), jnp.float32)
    for i, w2 in enumerate((e0_W2, e1_W2, e2_W2, e3_W2)):
        w2bd = w2bd.at[i * _H:(i + 1) * _H, i * _C:(i + 1) * _C].set(w2)
    b2_all = jnp.concatenate([e0_b2, e1_b2, e2_b2, e3_b2]).reshape(1, _CT)

    out_shape = [
        jax.ShapeDtypeStruct((4, _B, _C), jnp.float32),
        jax.ShapeDtypeStruct((_B, 4), jnp.float32),
        jax.ShapeDtypeStruct((_B, _C), jnp.float32),
        jax.ShapeDtypeStruct((1, 4), jnp.float32),
    ]
    vmem_spec = pl.BlockSpec(memory_space=pltpu.MemorySpace.VMEM)
    in_specs = [
        pl.BlockSpec((_B, _S), lambda k: (0, 0)),
        pl.BlockSpec((_B, _S), lambda k: (0, 0)),
        vmem_spec,
        pl.BlockSpec((_S2, _H), lambda k: (0, 0)),
        pl.BlockSpec((_S2, _H), lambda k: (0, 0)),
        pl.BlockSpec((_S2, _H), lambda k: (0, 0)),
        vmem_spec,
        pl.BlockSpec((1, _HT), lambda k: (0, 0)),
        pl.BlockSpec((_HT, _CT), lambda k: (0, 0)),
        pl.BlockSpec((1, _CT), lambda k: (0, 0)),
        pl.BlockSpec((1, _RW), lambda k: (0, 0)),
        pl.BlockSpec((_RW, _RW), lambda k: (0, 0)),
        pl.BlockSpec((1, _RW), lambda k: (0, 0)),
        pl.BlockSpec((_RW, 4), lambda k: (0, 0)),
        pl.BlockSpec((1, 4), lambda k: (0, 0)),
    ]
    out_specs = [
        pl.BlockSpec((4, _B, _C), lambda k: (0, 0, 0)),
        pl.BlockSpec((_B, 4), lambda k: (0, 0)),
        pl.BlockSpec((_B, _C), lambda k: (0, 0)),
        pl.BlockSpec((1, 4), lambda k: (0, 0)),
    ]

    eo, wgt, lg, loss = pl.pallas_call(
        _fused_body,
        grid=(1,),
        in_specs=in_specs,
        out_specs=out_specs,
        out_shape=out_shape,
        compiler_params=pltpu.CompilerParams(
            dimension_semantics=("arbitrary",)),
    )(eeg16, eog16, w0, w1, w2_, w3, rw16, b1_all, w2bd, b2_all,
      rw_b1.reshape(1, _RW), rw_W2, rw_b2.reshape(1, _RW), rw_Wo,
      rw_bo.reshape(1, 4))
    return eo, wgt, lg, loss.reshape(4)


# speedup vs baseline: 1.0470x; 1.0470x over previous
import functools

import jax
import jax.numpy as jnp
from jax.experimental import pallas as pl
from jax.experimental.pallas import tpu as pltpu

_B, _S, _H, _RW, _C = 256, 3000, 64, 256, 5
_S2 = 2 * _S
_HT = 4 * _H
_CT = 4 * _C


def _fused_body(eeg_ref, eog_ref, w0_ref, w1_ref, w2_ref, w3_ref, rw_ref,
                b1_ref, w2bd_ref, b2_ref, rwb1_ref, rwW2_ref, rwb2_ref,
                rwWo_ref, rwbo_ref,
                eo_ref, w_ref, lg_ref, loss_ref):
    dot = functools.partial(jnp.dot, preferred_element_type=jnp.float32)
    eeg = eeg_ref[...]
    eog = eog_ref[...]
    etop = jnp.concatenate(
        [w0_ref[0:_S, :], w1_ref[0:_S, :], w2_ref[0:_S, :], w3_ref[0:_S, :]],
        axis=1)
    ebot = jnp.concatenate(
        [w0_ref[_S:_S2, :], w1_ref[_S:_S2, :], w2_ref[_S:_S2, :],
         w3_ref[_S:_S2, :]], axis=1)
    a = dot(eeg, etop)
    b = dot(eeg, ebot)
    c = dot(eog, etop)
    d = dot(eog, ebot)
    r = dot(eeg, rw_ref[0:_S, :]) + dot(eog, rw_ref[_S:_S2, :])

    b1 = b1_ref[...]
    h_ee = jnp.maximum(a + b + b1, 0.0)
    h_oo = jnp.maximum(c + d + b1, 0.0)
    h_eo = jnp.maximum(a + d + b1, 0.0)
    w2 = w2bd_ref[...]
    b2 = b2_ref[...]
    out_ee = dot(h_ee, w2) + b2
    out_oo = dot(h_oo, w2) + b2
    out_eo = dot(h_eo, w2) + b2
    eo_ref[0, :, :] = out_ee[:, 0:5]
    eo_ref[1, :, :] = out_oo[:, 5:10]
    eo_ref[2, :, :] = out_eo[:, 10:15]
    eo_ref[3, :, :] = out_eo[:, 15:20]

    hr = jnp.maximum(r + rwb1_ref[...], 0.0)
    h2 = jnp.maximum(dot(hr, rwW2_ref[...]) + rwb2_ref[...], 0.0)
    rlog = dot(h2, rwWo_ref[...]) + rwbo_ref[...]
    m = jnp.max(rlog, axis=-1, keepdims=True)
    ex = jnp.exp(rlog - m)
    wgt = ex / jnp.sum(ex, axis=-1, keepdims=True)
    w_ref[...] = wgt
    lg_ref[...] = (out_ee[:, 0:5] * wgt[:, 0:1]
                   + out_oo[:, 5:10] * wgt[:, 1:2]
                   + out_eo[:, 10:15] * wgt[:, 2:3]
                   + out_eo[:, 15:20] * wgt[:, 3:4])

    def _mse(x, y):
        dlt = x - y
        return jnp.mean(dlt * dlt)

    u_eeg = -_mse(out_ee[:, 0:5], out_oo[:, 0:5])
    u_eog = -_mse(out_oo[:, 5:10], out_ee[:, 5:10])
    syn = -_mse(out_eo[:, 10:15], 0.5 * (out_ee[:, 10:15] + out_oo[:, 10:15]))
    red = _mse(out_ee[:, 15:20], out_oo[:, 15:20])
    loss_ref[...] = jnp.concatenate(
        [u_eeg.reshape(1, 1), u_eog.reshape(1, 1),
         syn.reshape(1, 1), red.reshape(1, 1)], axis=1)


def kernel(eeg, eog,
           e0_W1, e0_b1, e0_W2, e0_b2,
           e1_W1, e1_b1, e1_W2, e1_b2,
           e2_W1, e2_b1, e2_W2, e2_b2,
           e3_W1, e3_b1, e3_W2, e3_b2,
           rw_W1, rw_b1, rw_W2, rw_b2, rw_Wo, rw_bo):
    bf = jnp.bfloat16
    in_vmem = functools.partial(
        pltpu.with_memory_space_constraint,
        memory_space=pltpu.MemorySpace.VMEM)
    eeg16 = eeg.astype(bf)
    eog16 = eog.astype(bf)
    w0 = e0_W1.astype(bf)
    w1 = e1_W1.astype(bf)
    w2_ = e2_W1.astype(bf)
    w3 = e3_W1.astype(bf)
    rw16 = in_vmem(rw_W1.astype(bf))
    b1_all = jnp.concatenate([e0_b1, e1_b1, e2_b1, e3_b1]).reshape(1, _HT)
    w2bd = jnp.zeros((_HT, _CT), jnp.float32)
    for i, w2 in enumerate((e0_W2, e1_W2, e2_W2, e3_W2)):
        w2bd = w2bd.at[i * _H:(i + 1) * _H, i * _C:(i + 1) * _C].set(w2)
    b2_all = jnp.concatenate([e0_b2, e1_b2, e2_b2, e3_b2]).reshape(1, _CT)

    out_shape = [
        jax.ShapeDtypeStruct((4, _B, _C), jnp.float32),
        jax.ShapeDtypeStruct((_B, 4), jnp.float32),
        jax.ShapeDtypeStruct((_B, _C), jnp.float32),
        jax.ShapeDtypeStruct((1, 4), jnp.float32),
    ]
    vmem_spec = pl.BlockSpec(memory_space=pltpu.MemorySpace.VMEM)
    in_specs = [
        pl.BlockSpec((_B, _S), lambda k: (0, 0)),
        pl.BlockSpec((_B, _S), lambda k: (0, 0)),
        pl.BlockSpec((_S2, _H), lambda k: (0, 0)),
        pl.BlockSpec((_S2, _H), lambda k: (0, 0)),
        pl.BlockSpec((_S2, _H), lambda k: (0, 0)),
        pl.BlockSpec((_S2, _H), lambda k: (0, 0)),
        vmem_spec,
        pl.BlockSpec((1, _HT), lambda k: (0, 0)),
        pl.BlockSpec((_HT, _CT), lambda k: (0, 0)),
        pl.BlockSpec((1, _CT), lambda k: (0, 0)),
        pl.BlockSpec((1, _RW), lambda k: (0, 0)),
        pl.BlockSpec((_RW, _RW), lambda k: (0, 0)),
        pl.BlockSpec((1, _RW), lambda k: (0, 0)),
        pl.BlockSpec((_RW, 4), lambda k: (0, 0)),
        pl.BlockSpec((1, 4), lambda k: (0, 0)),
    ]
    out_specs = [
        pl.BlockSpec((4, _B, _C), lambda k: (0, 0, 0)),
        pl.BlockSpec((_B, 4), lambda k: (0, 0)),
        pl.BlockSpec((_B, _C), lambda k: (0, 0)),
        pl.BlockSpec((1, 4), lambda k: (0, 0)),
    ]

    eo, wgt, lg, loss = pl.pallas_call(
        _fused_body,
        grid=(1,),
        in_specs=in_specs,
        out_specs=out_specs,
        out_shape=out_shape,
        compiler_params=pltpu.CompilerParams(
            dimension_semantics=("arbitrary",)),
    )(eeg16, eog16, w0, w1, w2_, w3, rw16, b1_all, w2bd, b2_all,
      rw_b1.reshape(1, _RW), rw_W2, rw_b2.reshape(1, _RW), rw_Wo,
      rw_bo.reshape(1, 4))
    return eo, wgt, lg, loss.reshape(4)
